# baseline (speedup 1.0000x reference)
import functools

import jax
import jax.numpy as jnp
from jax import lax
from jax.experimental import pallas as pl
from jax.experimental.pallas import tpu as pltpu
from jax.experimental.pallas import tpu_sc as plsc

_L = 16


def _scores_sc(user_ids, item_ids, user_table, item_table):
    B = user_ids.shape[0]
    D = user_table.shape[1]
    info = plsc.get_sparse_core_info()
    nw = info.num_cores * info.num_subcores
    b_per_w = B // nw

    mesh = plsc.VectorSubcoreMesh(core_axis_name="c", subcore_axis_name="s")

    @functools.partial(
        pl.kernel,
        mesh=mesh,
        compiler_params=pltpu.CompilerParams(
            needs_layout_passes=False, use_tc_tiling_on_sc=False),
        out_type=jax.ShapeDtypeStruct((B,), jnp.float32),
        scratch_types=[
            pltpu.VMEM((b_per_w,), jnp.int32),
            pltpu.VMEM((b_per_w,), jnp.int32),
            pltpu.VMEM((b_per_w, D), jnp.float32),
            pltpu.VMEM((b_per_w, D), jnp.float32),
            pltpu.VMEM((_L * _L,), jnp.float32),
            pltpu.VMEM((b_per_w,), jnp.float32),
            pltpu.SemaphoreType.DMA,
            pltpu.SemaphoreType.DMA,
        ],
    )
    def k(uids_hbm, iids_hbm, utab_hbm, itab_hbm, out_hbm,
          uidx_v, iidx_v, urows_v, irows_v, tv, out_v, sem_u, sem_i):
        wid = lax.axis_index("s") * info.num_cores + lax.axis_index("c")
        base = wid * b_per_w
        pltpu.sync_copy(uids_hbm.at[pl.ds(base, b_per_w)], uidx_v)
        pltpu.sync_copy(iids_hbm.at[pl.ds(base, b_per_w)], iidx_v)
        cu = pltpu.async_copy(utab_hbm.at[uidx_v], urows_v, sem_u)
        ci = pltpu.async_copy(itab_hbm.at[iidx_v], irows_v, sem_i)
        cu.wait()
        ci.wait()

        lane = lax.iota(jnp.int32, _L)

        def blk_body(blk, carry):
            row0 = blk * _L
            for rj in range(_L):
                r = row0 + rj
                p = urows_v[r, pl.ds(0, _L)] * irows_v[r, pl.ds(0, _L)]
                for c in range(1, D // _L):
                    p = p + (urows_v[r, pl.ds(c * _L, _L)]
                             * irows_v[r, pl.ds(c * _L, _L)])
                plsc.store_scatter(tv, [lane * _L + rj], p)
            acc = tv[pl.ds(0, _L)]
            for l in range(1, _L):
                acc = acc + tv[pl.ds(l * _L, _L)]
            out_v[pl.ds(row0, _L)] = acc
            return carry

        lax.fori_loop(0, b_per_w // _L, blk_body, 0)
        pltpu.sync_copy(out_v, out_hbm.at[pl.ds(base, b_per_w)])

    return k(user_ids, item_ids, user_table, item_table)


def kernel(user_ids, item_ids, user_table, item_table):
    B = user_ids.shape[0]
    scores = _scores_sc(user_ids.astype(jnp.int32), item_ids.astype(jnp.int32),
                        user_table, item_table)
    return scores.reshape(B, 1)
